# baseline (speedup 1.0000x reference)
import functools

import jax
import jax.numpy as jnp
from jax import lax
from jax.experimental import pallas as pl
from jax.experimental.pallas import tpu as pltpu

M = 100000
B = 16384
D = 256
POS_DIM = 64

MB = 1024
BC = 1024
NBLK = 98
M_PAD = NBLK * MB

SC_NC = 2
SC_NS = 16
SC_NW = SC_NC * SC_NS
SC_CHUNK = 128
SC_CHUNKS_PER_W = 25
SC_ROWS_PER_W = SC_CHUNK * SC_CHUNKS_PER_W
M_SC = SC_NW * SC_ROWS_PER_W
TW = 384
TROWS = B + 16


def _agg_body(ids_ref, m1_ref, m2_ref, mean_ref, gidx_ref, has_ref):
    i = pl.program_id(0)
    node_col = i * MB + lax.broadcasted_iota(jnp.int32, (MB, 1), 0)

    def step(j, carry):
        counts, last, sums = carry
        idv = ids_ref[pl.ds(j, 1), :]
        mask = idv == node_col
        maskf = mask.astype(jnp.float32)
        counts = counts + jnp.sum(maskf, axis=1, keepdims=True)
        border = j * BC + lax.broadcasted_iota(jnp.int32, (1, BC), 1)
        last = jnp.maximum(
            last, jnp.max(jnp.where(mask, border, -1), axis=1, keepdims=True))
        pe = m1_ref[pl.ds(j * BC, BC), :] + m2_ref[pl.ds(j * BC, BC), :]
        sums = sums + jnp.dot(maskf, pe, preferred_element_type=jnp.float32)
        return counts, last, sums

    counts0 = jnp.zeros((MB, 1), jnp.float32)
    last0 = jnp.full((MB, 1), -1, jnp.int32)
    sums0 = jnp.zeros((MB, POS_DIM), jnp.float32)
    counts, last, sums = lax.fori_loop(0, B // BC, step,
                                       (counts0, last0, sums0))
    has = counts > 0.0
    mean_ref[...] = jnp.where(has, sums / jnp.maximum(counts, 1.0), 0.0)
    gidx_ref[...] = jnp.where(has, last, B)
    has_ref[...] = has.astype(jnp.float32)


def _segment_phase(ids2d, m1, m2):
    return pl.pallas_call(
        _agg_body,
        grid=(NBLK,),
        in_specs=[
            pl.BlockSpec((B // BC, BC), lambda i: (0, 0)),
            pl.BlockSpec((B, POS_DIM), lambda i: (0, 0)),
            pl.BlockSpec((B, POS_DIM), lambda i: (0, 0)),
        ],
        out_specs=[
            pl.BlockSpec((MB, POS_DIM), lambda i: (i, 0)),
            pl.BlockSpec((MB, 1), lambda i: (i, 0)),
            pl.BlockSpec((MB, 1), lambda i: (i, 0)),
        ],
        out_shape=[
            jax.ShapeDtypeStruct((M_PAD, POS_DIM), jnp.float32),
            jax.ShapeDtypeStruct((M_PAD, 1), jnp.int32),
            jax.ShapeDtypeStruct((M_PAD, 1), jnp.float32),
        ],
    )(ids2d, m1, m2)


def _make_sc_gather():
    from jax.experimental.pallas import tpu_sc as plsc

    mesh = plsc.VectorSubcoreMesh(core_axis_name="c", subcore_axis_name="s")

    @functools.partial(
        pl.kernel,
        mesh=mesh,
        out_type=jax.ShapeDtypeStruct((M_SC, TW), jnp.float32),
        scratch_types=[
            pltpu.VMEM((SC_CHUNK,), jnp.int32),
            pltpu.VMEM((SC_CHUNK, TW), jnp.float32),
            pltpu.SemaphoreType.DMA,
        ],
    )
    def sc_gather(table_hbm, gidx_hbm, out_hbm, idx_v, rows_v, sem):
        wid = lax.axis_index("s") * SC_NC + lax.axis_index("c")
        base = wid * SC_ROWS_PER_W

        def body(j, carry):
            off = base + j * SC_CHUNK
            pltpu.sync_copy(gidx_hbm.at[pl.ds(off, SC_CHUNK)], idx_v)
            pltpu.async_copy(table_hbm.at[idx_v], rows_v, sem).wait()
            pltpu.sync_copy(rows_v, out_hbm.at[pl.ds(off, SC_CHUNK)])
            return carry

        lax.fori_loop(0, SC_CHUNKS_PER_W, body, 0)

    return sc_gather


def kernel(messages, timestamps, node_ids):
    ids2d = node_ids.reshape(B // BC, BC)
    m1 = messages[:, D - (2 * POS_DIM + 1):D - (POS_DIM + 1)]
    m2 = messages[:, D - (POS_DIM + 1):D - 1]
    mean_pos, gidx, hasf = _segment_phase(ids2d, m1, m2)

    table = jnp.zeros((TROWS, TW), jnp.float32)
    table = table.at[:B, :D].set(messages)
    table = table.at[:B, D].set(timestamps)

    gidx_flat = jnp.concatenate(
        [gidx[:, 0], jnp.full((M_SC - M_PAD,), B, jnp.int32)])
    gathered = _make_sc_gather()(table, gidx_flat)

    unique_messages = jnp.concatenate(
        [gathered[:M, :D], mean_pos[:M]], axis=1)
    unique_timestamps = gathered[:M, D]
    has = hasf[:M, 0] > 0.0
    return unique_messages, unique_timestamps, has


# speedup vs baseline: 1.0063x; 1.0063x over previous
import functools

import jax
import jax.numpy as jnp
from jax import lax
from jax.experimental import pallas as pl
from jax.experimental.pallas import tpu as pltpu

M = 100000
B = 16384
D = 256
POS_DIM = 64

MB = 1024
BC = 1024
NBLK = 98
M_PAD = NBLK * MB

SC_NC = 2
SC_NS = 16
SC_NW = SC_NC * SC_NS
SC_CHUNK = 128
SC_CHUNKS_PER_W = 25
SC_ROWS_PER_W = SC_CHUNK * SC_CHUNKS_PER_W
M_SC = SC_NW * SC_ROWS_PER_W
TW = 384
TROWS = B + 16


def _agg_body(ids_ref, m1_ref, m2_ref, mean_ref, gidx_ref, has_ref):
    i = pl.program_id(0)
    node_col = i * MB + lax.broadcasted_iota(jnp.int32, (MB, 1), 0)

    def step(j, carry):
        counts, last, sums = carry
        idv = ids_ref[pl.ds(j, 1), :]
        mask = idv == node_col
        maskf = mask.astype(jnp.float32)
        counts = counts + jnp.sum(maskf, axis=1, keepdims=True)
        border = j * BC + lax.broadcasted_iota(jnp.int32, (1, BC), 1)
        last = jnp.maximum(
            last, jnp.max(jnp.where(mask, border, -1), axis=1, keepdims=True))
        pe = m1_ref[pl.ds(j * BC, BC), :] + m2_ref[pl.ds(j * BC, BC), :]
        sums = sums + jnp.dot(maskf, pe, preferred_element_type=jnp.float32)
        return counts, last, sums

    counts0 = jnp.zeros((MB, 1), jnp.float32)
    last0 = jnp.full((MB, 1), -1, jnp.int32)
    sums0 = jnp.zeros((MB, POS_DIM), jnp.float32)
    counts, last, sums = lax.fori_loop(0, B // BC, step,
                                       (counts0, last0, sums0))
    has = counts > 0.0
    mean_ref[...] = jnp.where(has, sums / jnp.maximum(counts, 1.0), 0.0)
    gidx_ref[...] = jnp.where(has, last, B)
    has_ref[...] = has.astype(jnp.float32)


def _segment_phase(ids2d, m1, m2):
    return pl.pallas_call(
        _agg_body,
        grid=(NBLK,),
        in_specs=[
            pl.BlockSpec((B // BC, BC), lambda i: (0, 0)),
            pl.BlockSpec((B, POS_DIM), lambda i: (0, 0)),
            pl.BlockSpec((B, POS_DIM), lambda i: (0, 0)),
        ],
        out_specs=[
            pl.BlockSpec((MB, POS_DIM), lambda i: (i, 0)),
            pl.BlockSpec((MB, 1), lambda i: (i, 0)),
            pl.BlockSpec((MB, 1), lambda i: (i, 0)),
        ],
        out_shape=[
            jax.ShapeDtypeStruct((M_PAD, POS_DIM), jnp.float32),
            jax.ShapeDtypeStruct((M_PAD, 1), jnp.int32),
            jax.ShapeDtypeStruct((M_PAD, 1), jnp.float32),
        ],
    )(ids2d, m1, m2)


def _make_sc_gather():
    from jax.experimental.pallas import tpu_sc as plsc

    mesh = plsc.VectorSubcoreMesh(core_axis_name="c", subcore_axis_name="s")

    @functools.partial(
        pl.kernel,
        mesh=mesh,
        out_type=jax.ShapeDtypeStruct((M_SC, TW), jnp.float32),
        scratch_types=[
            pltpu.VMEM((SC_ROWS_PER_W,), jnp.int32),
            pltpu.VMEM((SC_CHUNK, TW), jnp.float32),
            pltpu.VMEM((SC_CHUNK, TW), jnp.float32),
            pltpu.SemaphoreType.DMA,
            pltpu.SemaphoreType.DMA,
        ],
    )
    def sc_gather(table_hbm, gidx_hbm, out_hbm,
                  idx_all, rows_a, rows_b, sem_a, sem_b):
        wid = lax.axis_index("s") * SC_NC + lax.axis_index("c")
        base = wid * SC_ROWS_PER_W

        pltpu.sync_copy(gidx_hbm.at[pl.ds(base, SC_ROWS_PER_W)], idx_all)

        def gather_pair(p, carry):
            j0 = 2 * p
            j1 = 2 * p + 1
            cp_a = pltpu.async_copy(
                table_hbm.at[idx_all.at[pl.ds(j0 * SC_CHUNK, SC_CHUNK)]],
                rows_a, sem_a)
            cp_b = pltpu.async_copy(
                table_hbm.at[idx_all.at[pl.ds(j1 * SC_CHUNK, SC_CHUNK)]],
                rows_b, sem_b)
            cp_a.wait()
            pltpu.sync_copy(rows_a,
                            out_hbm.at[pl.ds(base + j0 * SC_CHUNK, SC_CHUNK)])
            cp_b.wait()
            pltpu.sync_copy(rows_b,
                            out_hbm.at[pl.ds(base + j1 * SC_CHUNK, SC_CHUNK)])
            return carry

        lax.fori_loop(0, SC_CHUNKS_PER_W // 2, gather_pair, 0)

        j_t = SC_CHUNKS_PER_W - 1
        cp_t = pltpu.async_copy(
            table_hbm.at[idx_all.at[pl.ds(j_t * SC_CHUNK, SC_CHUNK)]],
            rows_a, sem_a)
        cp_t.wait()
        pltpu.sync_copy(rows_a,
                        out_hbm.at[pl.ds(base + j_t * SC_CHUNK, SC_CHUNK)])

    return sc_gather


def kernel(messages, timestamps, node_ids):
    ids2d = node_ids.reshape(B // BC, BC)
    m1 = messages[:, D - (2 * POS_DIM + 1):D - (POS_DIM + 1)]
    m2 = messages[:, D - (POS_DIM + 1):D - 1]
    mean_pos, gidx, hasf = _segment_phase(ids2d, m1, m2)

    table = jnp.concatenate(
        [messages, timestamps[:, None],
         jnp.zeros((B, TW - D - 1), jnp.float32)], axis=1)
    table = jnp.concatenate(
        [table, jnp.zeros((TROWS - B, TW), jnp.float32)])
    gidx_flat = jnp.concatenate(
        [gidx[:, 0], jnp.full((M_SC - M_PAD,), B, jnp.int32)])
    gathered = _make_sc_gather()(table, gidx_flat)

    unique_messages = jnp.concatenate([gathered[:M, :D], mean_pos[:M]], axis=1)
    unique_timestamps = gathered[:M, D]
    has = hasf[:M, 0] > 0.0
    return unique_messages, unique_timestamps, has
